# baseline (speedup 1.0000x reference)
import jax
import jax.numpy as jnp
from jax.experimental import pallas as pl
from jax.experimental.pallas import tpu as pltpu

NUM_CLUSTERS = 128
DESC_DIM = 128
ANGLE_BINS = 8
POWER_ITERS = 12


def _sift_hist_kernel(x_ref, out_ref):
    img = x_ref[0, 0]
    gx = (jnp.roll(img, -1, axis=1) - jnp.roll(img, 1, axis=1)) * 0.5
    gy = (jnp.roll(img, -1, axis=0) - jnp.roll(img, 1, axis=0)) * 0.5
    mag = jnp.sqrt(gx * gx + gy * gy + 1e-12)
    a = -gx
    b = -gy
    q4 = b < 0.0
    a1 = jnp.where(q4, -a, a)
    b1 = jnp.where(q4, -b, b)
    q2 = a1 <= 0.0
    a2 = jnp.where(q2, b1, a1)
    b2 = jnp.where(q2, -a1, b1)
    q1 = b2 >= a2
    ang = (
        jnp.where(q4, 4, 0) + jnp.where(q2, 2, 0) + jnp.where(q1, 1, 0)
    ).astype(jnp.int32)
    ii = jax.lax.broadcasted_iota(jnp.int32, (64, 512), 0)
    jj = jax.lax.broadcasted_iota(jnp.int32, (64, 512), 1)
    S = (jj // 8 == ii).astype(jnp.float32)
    for a in range(ANGLE_BINS):
        Ma = jnp.where(ang == a, mag, 0.0)
        SM = jax.lax.dot_general(
            S, Ma, (((1,), (0,)), ((), ())), preferred_element_type=jnp.float32
        )
        Ha = jax.lax.dot_general(
            SM, S, (((1,), (1,)), ((), ())), preferred_element_type=jnp.float32
        )
        out_ref[0, a * 64:(a + 1) * 64, :] = Ha


def _vlad_kernel(descs_ref, cacc_ref, pops_ref, out_ref, rm_ref):
    B = descs_ref.shape[0]
    K, D = NUM_CLUSTERS, DESC_DIM
    centroids = cacc_ref[...] / pops_ref[...]
    ones_d = jnp.ones((1, D), jnp.float32)
    cn_row = jax.lax.dot_general(
        ones_d, centroids * centroids, (((1,), (1,)), ((), ())),
        preferred_element_type=jnp.float32,
    )
    ones_n = jnp.ones((256, 1), jnp.float32)
    kiota = jax.lax.broadcasted_iota(jnp.int32, (256, K), 1)
    for b in range(B):
        d = descs_ref[b]
        nrm = jnp.sqrt(jnp.sum(d * d, axis=1, keepdims=True))
        dn = d / (nrm + 1e-8)
        dc = jax.lax.dot_general(
            dn, centroids, (((1,), (1,)), ((), ())), preferred_element_type=jnp.float32
        )
        score = cn_row - 2.0 * dc
        minv = jnp.min(score, axis=1, keepdims=True)
        idx = jnp.min(jnp.where(score == minv, kiota, K + 1), axis=1, keepdims=True)
        A = (idx == kiota).astype(jnp.float32)
        desc_sums = jax.lax.dot_general(
            A, dn, (((0,), (0,)), ((), ())), preferred_element_type=jnp.float32
        )
        pops_col = jax.lax.dot_general(
            A, ones_n, (((0,), (0,)), ((), ())), preferred_element_type=jnp.float32
        )
        rm_ref[b] = centroids * pops_col - desc_sums
    Rm = rm_ref[...]
    v = jnp.ones((B, D), jnp.float32) + jax.lax.broadcasted_iota(
        jnp.int32, (B, D), 1
    ).astype(jnp.float32) * 1e-3
    v = v / jnp.sqrt(jnp.sum(v * v, axis=1, keepdims=True))

    def body(i, v):
        w = jnp.sum(Rm * v[:, None, :], axis=2)
        u = jnp.sum(Rm * w[:, :, None], axis=1)
        return u / (jnp.sqrt(jnp.sum(u * u, axis=1, keepdims=True)) + 1e-30)

    v = jax.lax.fori_loop(0, POWER_ITERS, body, v)
    w = jnp.sum(Rm * v[:, None, :], axis=2)
    sigma = jnp.sqrt(jnp.sum(w * w, axis=1, keepdims=True))
    jj = jax.lax.broadcasted_iota(jnp.int32, (D, D), 0)
    dd = jax.lax.broadcasted_iota(jnp.int32, (D, D), 1)
    orig = ((jj // 4) % 4) * 32 + (jj % 4) * 8 + jj // 16
    P = (dd == orig).astype(jnp.float32)
    for b in range(B):
        out_ref[b] = jax.lax.dot_general(
            Rm[b] / sigma[b], P, (((1,), (0,)), ((), ())),
            preferred_element_type=jnp.float32,
        )


@jax.jit
def kernel(x, centroids_acc, populations):
    B = x.shape[0]
    hist = pl.pallas_call(
        _sift_hist_kernel,
        grid=(B,),
        in_specs=[pl.BlockSpec((1, 1, 512, 512), lambda b: (b, 0, 0, 0))],
        out_specs=pl.BlockSpec((1, ANGLE_BINS * 64, 64), lambda b: (b, 0, 0)),
        out_shape=jax.ShapeDtypeStruct((B, ANGLE_BINS * 64, 64), jnp.float32),
    )(x)
    descs_p = (
        hist.reshape(B, ANGLE_BINS, 16, 4, 16, 4)
        .transpose(0, 2, 4, 1, 3, 5)
        .reshape(B, 256, DESC_DIM)
    )
    j = jnp.arange(DESC_DIM)
    perm = ((j // 4) % 4) * 32 + (j % 4) * 8 + j // 16
    cacc_p = centroids_acc[:, perm]
    out = pl.pallas_call(
        _vlad_kernel,
        out_shape=jax.ShapeDtypeStruct((B, NUM_CLUSTERS, DESC_DIM), jnp.float32),
        scratch_shapes=[pltpu.VMEM((B, NUM_CLUSTERS, DESC_DIM), jnp.float32)],
    )(descs_p, cacc_p, populations.reshape(NUM_CLUSTERS, 1))
    return out
